# baseline (speedup 1.0000x reference)
import functools

import jax
import jax.numpy as jnp
from jax import lax
from jax.experimental import pallas as pl
from jax.experimental.pallas import tpu as pltpu
from jax.experimental.pallas import tpu_sc as plsc

_L = 16


def _newton_rsqrt(v):
    i = lax.bitcast_convert_type(v, jnp.int32)
    i = 0x5F3759DF - lax.shift_right_arithmetic(i, 1)
    y = lax.bitcast_convert_type(i, jnp.float32)
    for _ in range(4):
        y = y * (1.5 - 0.5 * v * y * y)
    return y


def _lane_sum(v):
    lanes = lax.iota(jnp.int32, _L)
    for k in (1, 2, 4, 8):
        perm = jnp.bitwise_xor(lanes, k)
        v = v + v.at[perm].get(mode="promise_in_bounds")
    return v


def _renorm_rows(rows_ref, out_ref, n_rows, d, half):
    for r in range(n_rows):
        acc = jnp.zeros((_L,), jnp.float32)
        for c in range(d // _L):
            ch = rows_ref[r, pl.ds(c * _L, _L)]
            acc = acc + ch * ch
        s2 = _lane_sum(acc)
        y = _newton_rsqrt(s2)
        norm = s2 * y
        scale = jnp.where(norm > 1.0, 1.0 / (norm + 1e-7), 1.0)
        for c in range(half // _L):
            out_ref[r, pl.ds(c * _L, _L)] = rows_ref[r, pl.ds(c * _L, _L)] * scale


def _sc_table_prep(seq_s, per_table, np16, *, s, p, d, half):
    info = plsc.get_sparse_core_info()
    nc, ns = info.num_cores, info.num_subcores
    seq_rows = 8
    n_seq_tiles = s // seq_rows
    per_rows = _L
    n_per_tiles = p // per_rows
    tab_rows = per_table.shape[0]

    @functools.partial(
        pl.kernel,
        mesh=plsc.VectorSubcoreMesh(core_axis_name="c", subcore_axis_name="s"),
        out_type=(
            jax.ShapeDtypeStruct((s, half), jnp.float32),
            jax.ShapeDtypeStruct((p, half), jnp.float32),
        ),
        scratch_types=[
            pltpu.VMEM((seq_rows, d), jnp.float32),
            pltpu.VMEM((seq_rows, half), jnp.float32),
            pltpu.VMEM((per_rows, d), jnp.float32),
            pltpu.VMEM((per_rows, half), jnp.float32),
            pltpu.VMEM((_L,), jnp.int32),
            pltpu.VMEM((_L,), jnp.int32),
            pltpu.SemaphoreType.DMA,
        ],
    )
    def prep(seq_hbm, per_hbm, np_hbm, seq_out, per_out,
             srows_v, sout_v, prows_v, pout_v, np_v, idx_v, sem):
        wid = lax.axis_index("s") * nc + lax.axis_index("c")

        @pl.when(wid < n_seq_tiles)
        def _seq():
            base = wid * seq_rows
            pltpu.sync_copy(seq_hbm.at[pl.ds(base, seq_rows)], srows_v)
            _renorm_rows(srows_v, sout_v, seq_rows, d, half)
            pltpu.sync_copy(sout_v, seq_out.at[pl.ds(base, seq_rows)])

        @pl.when((wid >= n_seq_tiles) & (wid < n_seq_tiles + n_per_tiles))
        def _person():
            pbase = (wid - n_seq_tiles) * per_rows
            pltpu.sync_copy(np_hbm, np_v)
            rowids = pbase + lax.iota(jnp.int32, _L)
            idx = jnp.minimum(rowids, np_v[...] - 1)
            idx = jnp.minimum(jnp.maximum(idx, 0), tab_rows - 1)
            idx_v[...] = idx
            pltpu.async_copy(per_hbm.at[idx_v], prows_v, sem).wait()
            _renorm_rows(prows_v, pout_v, per_rows, d, half)
            pltpu.sync_copy(pout_v, per_out.at[pl.ds(pbase, per_rows)])

    return prep(seq_s, per_table, np16)


def _enc_add_kernel(seqh_ref, perh_ref, x_ref, o_ref, enc_ref):
    @pl.when(pl.program_id(0) == 0)
    def _build_enc():
        sh = seqh_ref[...]
        ph = perh_ref[...]
        s, h = sh.shape
        p = ph.shape[0]
        enc_ref[...] = jnp.concatenate(
            [
                jnp.broadcast_to(sh[:, None, :], (s, p, h)),
                jnp.broadcast_to(ph[None, :, :], (s, p, h)),
            ],
            axis=-1,
        )

    o_ref[...] = x_ref[...] + enc_ref[...]


def kernel(x, seq_encoding, person_encoding, num_people):
    b, s, p, d = x.shape
    half = d // 2
    bb = next((c for c in (4, 2) if b % c == 0), 1)

    np16 = jnp.full((_L,), jnp.asarray(num_people, jnp.int32))
    seq_half, per_half = _sc_table_prep(
        seq_encoding, person_encoding, np16, s=s, p=p, d=d, half=half
    )

    return pl.pallas_call(
        _enc_add_kernel,
        grid=(b // bb,),
        in_specs=[
            pl.BlockSpec((s, half), lambda j: (0, 0)),
            pl.BlockSpec((p, half), lambda j: (0, 0)),
            pl.BlockSpec((bb, s, p, d), lambda j: (j, 0, 0, 0)),
        ],
        out_specs=pl.BlockSpec((bb, s, p, d), lambda j: (j, 0, 0, 0)),
        out_shape=jax.ShapeDtypeStruct((b, s, p, d), x.dtype),
        scratch_shapes=[pltpu.VMEM((s, p, d), x.dtype)],
    )(seq_half, per_half, x)


# speedup vs baseline: 1.0177x; 1.0177x over previous
import functools

import jax
import jax.numpy as jnp
from jax import lax
from jax.experimental import pallas as pl
from jax.experimental.pallas import tpu as pltpu
from jax.experimental.pallas import tpu_sc as plsc

_L = 16


def _newton_rsqrt(v):
    i = lax.bitcast_convert_type(v, jnp.int32)
    i = 0x5F3759DF - lax.shift_right_arithmetic(i, 1)
    y = lax.bitcast_convert_type(i, jnp.float32)
    for _ in range(4):
        y = y * (1.5 - 0.5 * v * y * y)
    return y


def _lane_sum(v):
    lanes = lax.iota(jnp.int32, _L)
    for k in (1, 2, 4, 8):
        perm = jnp.bitwise_xor(lanes, k)
        v = v + v.at[perm].get(mode="promise_in_bounds")
    return v


def _renorm_rows(rows_ref, out_ref, n_rows, d, half):
    for r in range(n_rows):
        acc = jnp.zeros((_L,), jnp.float32)
        for c in range(d // _L):
            ch = rows_ref[r, pl.ds(c * _L, _L)]
            acc = acc + ch * ch
        s2 = _lane_sum(acc)
        y = _newton_rsqrt(s2)
        norm = s2 * y
        scale = jnp.where(norm > 1.0, 1.0 / (norm + 1e-7), 1.0)
        for c in range(half // _L):
            out_ref[r, pl.ds(c * _L, _L)] = rows_ref[r, pl.ds(c * _L, _L)] * scale


def _sc_table_prep(seq_table, per_table, np16, *, s, p, d, half):
    info = plsc.get_sparse_core_info()
    nc = info.num_cores
    seq_rows = 8
    n_seq_tiles = s // seq_rows
    per_rows = _L
    n_per_tiles = p // per_rows
    tab_rows = per_table.shape[0]

    @functools.partial(
        pl.kernel,
        mesh=plsc.VectorSubcoreMesh(core_axis_name="c", subcore_axis_name="s"),
        out_type=(
            jax.ShapeDtypeStruct((s, half), jnp.float32),
            jax.ShapeDtypeStruct((p, half), jnp.float32),
        ),
        scratch_types=[
            pltpu.VMEM((seq_rows, d), jnp.float32),
            pltpu.VMEM((seq_rows, half), jnp.float32),
            pltpu.VMEM((per_rows, d), jnp.float32),
            pltpu.VMEM((per_rows, half), jnp.float32),
            pltpu.VMEM((_L,), jnp.int32),
            pltpu.VMEM((_L,), jnp.int32),
            pltpu.SemaphoreType.DMA,
        ],
    )
    def prep(seq_hbm, per_hbm, np_hbm, seq_out, per_out,
             srows_v, sout_v, prows_v, pout_v, np_v, idx_v, sem):
        wid = lax.axis_index("s") * nc + lax.axis_index("c")

        @pl.when(wid < n_seq_tiles)
        def _seq():
            base = wid * seq_rows
            pltpu.sync_copy(seq_hbm.at[pl.ds(base, seq_rows)], srows_v)
            _renorm_rows(srows_v, sout_v, seq_rows, d, half)
            pltpu.sync_copy(sout_v, seq_out.at[pl.ds(base, seq_rows)])

        @pl.when((wid >= n_seq_tiles) & (wid < n_seq_tiles + n_per_tiles))
        def _person():
            pbase = (wid - n_seq_tiles) * per_rows
            pltpu.sync_copy(np_hbm, np_v)
            rowids = pbase + lax.iota(jnp.int32, _L)
            idx = jnp.minimum(rowids, np_v[...] - 1)
            idx = jnp.minimum(jnp.maximum(idx, 0), tab_rows - 1)
            idx_v[...] = idx
            pltpu.async_copy(per_hbm.at[idx_v], prows_v, sem).wait()
            _renorm_rows(prows_v, pout_v, per_rows, d, half)
            pltpu.sync_copy(pout_v, per_out.at[pl.ds(pbase, per_rows)])

    return prep(seq_table, per_table, np16)


def _tc_head_kernel(np_ref, seq_ref, per_ref, x_ref, o_ref, enc_ref, *, half):
    @pl.when(pl.program_id(0) == 0)
    def _build_enc():
        sq = seq_ref[...]
        snorm = jnp.sqrt(jnp.sum(sq * sq, axis=-1, keepdims=True))
        sscale = jnp.where(snorm > 1.0, 1.0 / (snorm + 1e-7), 1.0)
        sq_half = (sq * sscale)[:, :half]

        pt = per_ref[...]
        num_people = np_ref[0]
        p_cap = pt.shape[0]
        last = per_ref[pl.ds(jnp.minimum(num_people - 1, p_cap - 1), 1), :]
        pidx = lax.broadcasted_iota(jnp.int32, (p_cap, 1), 0)
        psel = jnp.where(pidx < num_people, pt, last)
        pnorm = jnp.sqrt(jnp.sum(psel * psel, axis=-1, keepdims=True))
        pscale = jnp.where(pnorm > 1.0, 1.0 / (pnorm + 1e-7), 1.0)
        pt_half = (psel * pscale)[:, :half]

        s = sq.shape[0]
        enc_ref[...] = jnp.concatenate(
            [
                jnp.broadcast_to(sq_half[:, None, :], (s, p_cap, half)),
                jnp.broadcast_to(pt_half[None, :, :], (s, p_cap, half)),
            ],
            axis=-1,
        )

    o_ref[...] = x_ref[...] + enc_ref[...]


def _tc_tail_kernel(prev_ref, seqh_ref, perh_ref, x_ref, o_ref, enc_ref):
    del prev_ref

    @pl.when(pl.program_id(0) == 0)
    def _build_enc():
        sh = seqh_ref[...]
        ph = perh_ref[...]
        s, h = sh.shape
        p = ph.shape[0]
        enc_ref[...] = jnp.concatenate(
            [
                jnp.broadcast_to(sh[:, None, :], (s, p, h)),
                jnp.broadcast_to(ph[None, :, :], (s, p, h)),
            ],
            axis=-1,
        )

    o_ref[...] = x_ref[...] + enc_ref[...]


def kernel(x, seq_encoding, person_encoding, num_people):
    b, s, p, d = x.shape
    half = d // 2
    bb = next((c for c in (4, 2) if b % c == 0), 1)
    head = 2 * bb if b > 2 * bb else bb
    n_head = head // bb
    n_tail = (b - head) // bb

    np16 = jnp.full((_L,), jnp.asarray(num_people, jnp.int32))
    np1 = jnp.asarray(num_people, jnp.int32).reshape((1,))
    seq_half, per_half = _sc_table_prep(
        seq_encoding, person_encoding, np16, s=s, p=p, d=d, half=half
    )

    out_head = pl.pallas_call(
        functools.partial(_tc_head_kernel, half=half),
        grid=(n_head,),
        in_specs=[
            pl.BlockSpec(memory_space=pltpu.SMEM),
            pl.BlockSpec((s, d), lambda j: (0, 0)),
            pl.BlockSpec((p, d), lambda j: (0, 0)),
            pl.BlockSpec((bb, s, p, d), lambda j: (j, 0, 0, 0)),
        ],
        out_specs=pl.BlockSpec((bb, s, p, d), lambda j: (j, 0, 0, 0)),
        out_shape=jax.ShapeDtypeStruct((b, s, p, d), x.dtype),
        scratch_shapes=[pltpu.VMEM((s, p, d), x.dtype)],
    )(np1, seq_encoding[:s], person_encoding[:p], x)

    if n_tail == 0:
        return out_head

    off = n_head
    return pl.pallas_call(
        _tc_tail_kernel,
        grid=(n_tail,),
        in_specs=[
            pl.BlockSpec(memory_space=pl.ANY),
            pl.BlockSpec((s, half), lambda j: (0, 0)),
            pl.BlockSpec((p, half), lambda j: (0, 0)),
            pl.BlockSpec((bb, s, p, d), lambda j: (j + off, 0, 0, 0)),
        ],
        out_specs=pl.BlockSpec((bb, s, p, d), lambda j: (j + off, 0, 0, 0)),
        out_shape=jax.ShapeDtypeStruct((b, s, p, d), x.dtype),
        scratch_shapes=[pltpu.VMEM((s, p, d), x.dtype)],
        input_output_aliases={0: 0},
    )(out_head, seq_half, per_half, x)


# speedup vs baseline: 1.1368x; 1.1170x over previous
import functools

import jax
import jax.numpy as jnp
from jax import lax
from jax.experimental import pallas as pl
from jax.experimental.pallas import tpu as pltpu
from jax.experimental.pallas import tpu_sc as plsc

_L = 16


def _newton_rsqrt(v):
    i = lax.bitcast_convert_type(v, jnp.int32)
    i = 0x5F3759DF - lax.shift_right_arithmetic(i, 1)
    y = lax.bitcast_convert_type(i, jnp.float32)
    for _ in range(4):
        y = y * (1.5 - 0.5 * v * y * y)
    return y


def _lane_sum(v):
    lanes = lax.iota(jnp.int32, _L)
    for k in (1, 2, 4, 8):
        perm = jnp.bitwise_xor(lanes, k)
        v = v + v.at[perm].get(mode="promise_in_bounds")
    return v


def _renorm_rows(rows_ref, out_ref, n_rows, d, half):
    for r in range(n_rows):
        acc = jnp.zeros((_L,), jnp.float32)
        for c in range(d // _L):
            ch = rows_ref[r, pl.ds(c * _L, _L)]
            acc = acc + ch * ch
        s2 = _lane_sum(acc)
        y = _newton_rsqrt(s2)
        norm = s2 * y
        scale = jnp.where(norm > 1.0, 1.0 / (norm + 1e-7), 1.0)
        for c in range(half // _L):
            out_ref[r, pl.ds(c * _L, _L)] = rows_ref[r, pl.ds(c * _L, _L)] * scale


def _sc_table_prep(seq_table, per_table, np16, *, s, p, d, half):
    info = plsc.get_sparse_core_info()
    nc = info.num_cores
    seq_rows = 8
    n_seq_tiles = s // seq_rows
    per_rows = _L
    n_per_tiles = p // per_rows
    tab_rows = per_table.shape[0]

    @functools.partial(
        pl.kernel,
        mesh=plsc.VectorSubcoreMesh(core_axis_name="c", subcore_axis_name="s"),
        out_type=(
            jax.ShapeDtypeStruct((s, half), jnp.float32),
            jax.ShapeDtypeStruct((p, half), jnp.float32),
        ),
        scratch_types=[
            pltpu.VMEM((seq_rows, d), jnp.float32),
            pltpu.VMEM((seq_rows, half), jnp.float32),
            pltpu.VMEM((per_rows, d), jnp.float32),
            pltpu.VMEM((per_rows, half), jnp.float32),
            pltpu.VMEM((_L,), jnp.int32),
            pltpu.VMEM((_L,), jnp.int32),
            pltpu.SemaphoreType.DMA,
        ],
    )
    def prep(seq_hbm, per_hbm, np_hbm, seq_out, per_out,
             srows_v, sout_v, prows_v, pout_v, np_v, idx_v, sem):
        wid = lax.axis_index("s") * nc + lax.axis_index("c")

        @pl.when(wid < n_seq_tiles)
        def _seq():
            base = wid * seq_rows
            pltpu.sync_copy(seq_hbm.at[pl.ds(base, seq_rows)], srows_v)
            _renorm_rows(srows_v, sout_v, seq_rows, d, half)
            pltpu.sync_copy(sout_v, seq_out.at[pl.ds(base, seq_rows)])

        @pl.when((wid >= n_seq_tiles) & (wid < n_seq_tiles + n_per_tiles))
        def _person():
            pbase = (wid - n_seq_tiles) * per_rows
            pltpu.sync_copy(np_hbm, np_v)
            rowids = pbase + lax.iota(jnp.int32, _L)
            idx = jnp.minimum(rowids, np_v[...] - 1)
            idx = jnp.minimum(jnp.maximum(idx, 0), tab_rows - 1)
            idx_v[...] = idx
            pltpu.async_copy(per_hbm.at[idx_v], prows_v, sem).wait()
            _renorm_rows(prows_v, pout_v, per_rows, d, half)
            pltpu.sync_copy(pout_v, per_out.at[pl.ds(pbase, per_rows)])

    return prep(seq_table, per_table, np16)


def _tc_head_kernel(np_ref, seq_ref, per_ref, x_ref, o_ref, enc_ref, *, half):
    @pl.when(pl.program_id(0) == 0)
    def _build_enc():
        sq = seq_ref[...]
        snorm = jnp.sqrt(jnp.sum(sq * sq, axis=-1, keepdims=True))
        sscale = jnp.where(snorm > 1.0, 1.0 / (snorm + 1e-7), 1.0)
        sq_half = (sq * sscale)[:, :half]

        pt = per_ref[...]
        num_people = np_ref[0]
        p_cap = pt.shape[0]
        last = per_ref[pl.ds(jnp.minimum(num_people - 1, p_cap - 1), 1), :]
        pidx = lax.broadcasted_iota(jnp.int32, (p_cap, 1), 0)
        psel = jnp.where(pidx < num_people, pt, last)
        pnorm = jnp.sqrt(jnp.sum(psel * psel, axis=-1, keepdims=True))
        pscale = jnp.where(pnorm > 1.0, 1.0 / (pnorm + 1e-7), 1.0)
        pt_half = (psel * pscale)[:, :half]

        s = sq.shape[0]
        enc_ref[...] = jnp.concatenate(
            [
                jnp.broadcast_to(sq_half[:, None, :], (s, p_cap, half)),
                jnp.broadcast_to(pt_half[None, :, :], (s, p_cap, half)),
            ],
            axis=-1,
        )

    o_ref[...] = x_ref[...] + enc_ref[...]


def _tc_tail_kernel(prev_ref, seqh_ref, perh_ref, x_ref, o_ref, enc_ref):
    del prev_ref

    @pl.when(pl.program_id(0) == 0)
    def _build_enc():
        sh = seqh_ref[...]
        ph = perh_ref[...]
        s, h = sh.shape
        p = ph.shape[0]
        enc_ref[...] = jnp.concatenate(
            [
                jnp.broadcast_to(sh[:, None, :], (s, p, h)),
                jnp.broadcast_to(ph[None, :, :], (s, p, h)),
            ],
            axis=-1,
        )

    o_ref[...] = x_ref[...] + enc_ref[...]


def kernel(x, seq_encoding, person_encoding, num_people):
    b, s, p, d = x.shape
    half = d // 2
    bb = next((c for c in (4, 2) if b % c == 0), 1)
    head = 2 * bb if b > 2 * bb else bb
    n_head = head // bb
    n_tail = (b - head) // bb

    np16 = jnp.full((_L,), jnp.asarray(num_people, jnp.int32))
    np1 = jnp.asarray(num_people, jnp.int32).reshape((1,))
    seq_half, per_half = _sc_table_prep(
        seq_encoding, person_encoding, np16, s=s, p=p, d=d, half=half
    )

    out_head = pl.pallas_call(
        functools.partial(_tc_head_kernel, half=half),
        grid=(n_head,),
        in_specs=[
            pl.BlockSpec(memory_space=pltpu.SMEM),
            pl.BlockSpec((s, d), lambda j: (0, 0)),
            pl.BlockSpec((p, d), lambda j: (0, 0)),
            pl.BlockSpec((bb, s, p, d), lambda j: (j, 0, 0, 0)),
        ],
        out_specs=pl.BlockSpec((bb, s, p, d), lambda j: (j, 0, 0, 0)),
        out_shape=jax.ShapeDtypeStruct((b, s, p, d), x.dtype),
        scratch_shapes=[pltpu.VMEM((s, p, d), x.dtype)],
    )(np1, seq_encoding[:s], person_encoding[:p], x)

    if n_tail == 0:
        return out_head

    off = n_head
    del seq_half, per_half

    def _tc_tail_ctl(prev_ref, np_ref, seq_ref, per_ref, x_ref, o_ref, enc_ref):
        del prev_ref
        _tc_head_kernel(np_ref, seq_ref, per_ref, x_ref, o_ref, enc_ref, half=half)

    return pl.pallas_call(
        _tc_tail_ctl,
        grid=(n_tail,),
        in_specs=[
            pl.BlockSpec(memory_space=pl.ANY),
            pl.BlockSpec(memory_space=pltpu.SMEM),
            pl.BlockSpec((s, d), lambda j: (0, 0)),
            pl.BlockSpec((p, d), lambda j: (0, 0)),
            pl.BlockSpec((bb, s, p, d), lambda j: (j + off, 0, 0, 0)),
        ],
        out_specs=pl.BlockSpec((bb, s, p, d), lambda j: (j + off, 0, 0, 0)),
        out_shape=jax.ShapeDtypeStruct((b, s, p, d), x.dtype),
        scratch_shapes=[pltpu.VMEM((s, p, d), x.dtype)],
        input_output_aliases={0: 0},
    )(out_head, np1, seq_encoding[:s], person_encoding[:p], x)
